# baseline (speedup 1.0000x reference)
import functools

import jax
import jax.numpy as jnp
from jax.experimental import pallas as pl
from jax.experimental.pallas import tpu as pltpu


def _fp_body(x1_ref, x2t_ref, p1_ref, p2_ref, W1_ref, b1_ref, W2_ref, b2_ref,
             out_ref, q2_ref, *, N2, C2, RB):
    j = pl.program_id(1)

    @pl.when(j == 0)
    def _():
        q2_ref[...] = jnp.dot(p2_ref[...], W1_ref[:C2, :],
                              preferred_element_type=jnp.float32)

    x1 = x1_ref[...]
    x2t = x2t_ref[...]

    x1sq = (x1[:, 0:1] * x1[:, 0:1] + x1[:, 1:2] * x1[:, 1:2]
            + x1[:, 2:3] * x1[:, 2:3])
    x2sq = (x2t[0:1, :] * x2t[0:1, :] + x2t[1:2, :] * x2t[1:2, :]
            + x2t[2:3, :] * x2t[2:3, :])
    cross = jnp.dot(x1, x2t, preferred_element_type=jnp.float32,
                    precision=jax.lax.Precision.HIGHEST)
    d = (x1sq - 2.0 * cross) + x2sq

    inf = jnp.float32(jnp.inf)
    m1 = jnp.min(d, axis=1, keepdims=True)
    d1 = jnp.where(d == m1, inf, d)
    m2 = jnp.min(d1, axis=1, keepdims=True)
    d2 = jnp.where(d1 == m2, inf, d1)
    m3 = jnp.min(d2, axis=1, keepdims=True)

    r = [1.0 / jnp.maximum(m, 1e-10) for m in (m1, m2, m3)]
    norm = (r[0] + r[1]) + r[2]
    w = [ri / norm for ri in r]

    S = (jnp.where(d == m1, w[0], 0.0)
         + jnp.where(d == m2, w[1], 0.0)
         + jnp.where(d == m3, w[2], 0.0))

    interp_h = jnp.dot(S, q2_ref[...], preferred_element_type=jnp.float32)
    p1h = jnp.dot(p1_ref[...], W1_ref[C2:, :],
                  preferred_element_type=jnp.float32)
    h1 = jnp.maximum(interp_h + p1h + b1_ref[...], 0.0)
    h2 = jnp.dot(h1, W2_ref[...], preferred_element_type=jnp.float32)
    out_ref[...] = jnp.maximum(h2 + b2_ref[...], 0.0)


def kernel(xyz1, xyz2, points1, points2, W1, b1, W2, b2):
    B, N1, _ = xyz1.shape
    N2 = xyz2.shape[1]
    C1 = points1.shape[2]
    C2 = points2.shape[2]
    H = W1.shape[1]
    O = W2.shape[1]

    RB = 1024
    NB = N1 // RB

    x2t = jnp.transpose(xyz2, (0, 2, 1))
    b1r = b1.reshape(1, H)
    b2r = b2.reshape(1, O)

    body = functools.partial(_fp_body, N2=N2, C2=C2, RB=RB)
    out = pl.pallas_call(
        body,
        grid=(B, NB),
        in_specs=[
            pl.BlockSpec((None, RB, 3), lambda b, j: (b, j, 0)),
            pl.BlockSpec((None, 3, N2), lambda b, j: (b, 0, 0)),
            pl.BlockSpec((None, RB, C1), lambda b, j: (b, j, 0)),
            pl.BlockSpec((None, N2, C2), lambda b, j: (b, 0, 0)),
            pl.BlockSpec((C1 + C2, H), lambda b, j: (0, 0)),
            pl.BlockSpec((1, H), lambda b, j: (0, 0)),
            pl.BlockSpec((H, O), lambda b, j: (0, 0)),
            pl.BlockSpec((1, O), lambda b, j: (0, 0)),
        ],
        out_specs=pl.BlockSpec((None, RB, O), lambda b, j: (b, j, 0)),
        out_shape=jax.ShapeDtypeStruct((B, N1, O), jnp.float32),
        scratch_shapes=[pltpu.VMEM((N2, H), jnp.float32)],
    )(xyz1, x2t, points1, points2, W1, b1r, W2, b2r)
    return out


# speedup vs baseline: 1.4884x; 1.4884x over previous
import functools

import jax
import jax.numpy as jnp
from jax.experimental import pallas as pl
from jax.experimental.pallas import tpu as pltpu


def _fp_body(x1_ref, x2t_ref, p1_ref, p2_ref, W1_ref, b1_ref, W2_ref, b2_ref,
             out_ref, q2_ref, *, N2, C2, RB):
    j = pl.program_id(1)

    @pl.when(j == 0)
    def _():
        q2_ref[...] = jnp.dot(p2_ref[...], W1_ref[:C2, :],
                              preferred_element_type=jnp.float32)

    x1 = x1_ref[...]
    x2t = x2t_ref[...]

    d0 = x1[:, 0:1] - x2t[0:1, :]
    d1c = x1[:, 1:2] - x2t[1:2, :]
    d2c = x1[:, 2:3] - x2t[2:3, :]
    d = (d0 * d0 + d1c * d1c) + d2c * d2c

    inf = jnp.float32(jnp.inf)
    m1 = jnp.min(d, axis=1, keepdims=True)
    eq1 = d == m1
    d1 = jnp.where(eq1, inf, d)
    m2 = jnp.min(d1, axis=1, keepdims=True)
    eq2 = d1 == m2
    d2 = jnp.where(eq2, inf, d1)
    m3 = jnp.min(d2, axis=1, keepdims=True)
    eq3 = d2 == m3

    r = [1.0 / jnp.maximum(m, 1e-10) for m in (m1, m2, m3)]
    norm = (r[0] + r[1]) + r[2]
    w = [ri / norm for ri in r]

    S = jnp.where(eq1, w[0],
                  jnp.where(eq2, w[1],
                            jnp.where(eq3, w[2], 0.0)))

    interp_h = jnp.dot(S, q2_ref[...], preferred_element_type=jnp.float32)
    p1h = jnp.dot(p1_ref[...], W1_ref[C2:, :],
                  preferred_element_type=jnp.float32)
    h1 = jnp.maximum(interp_h + p1h + b1_ref[...], 0.0)
    h2 = jnp.dot(h1, W2_ref[...], preferred_element_type=jnp.float32)
    out_ref[...] = jnp.maximum(h2 + b2_ref[...], 0.0)


def kernel(xyz1, xyz2, points1, points2, W1, b1, W2, b2):
    B, N1, _ = xyz1.shape
    N2 = xyz2.shape[1]
    C1 = points1.shape[2]
    C2 = points2.shape[2]
    H = W1.shape[1]
    O = W2.shape[1]

    RB = 1024
    NB = N1 // RB

    x2t = jnp.transpose(xyz2, (0, 2, 1))
    b1r = b1.reshape(1, H)
    b2r = b2.reshape(1, O)

    body = functools.partial(_fp_body, N2=N2, C2=C2, RB=RB)
    out = pl.pallas_call(
        body,
        grid=(B, NB),
        in_specs=[
            pl.BlockSpec((None, RB, 3), lambda b, j: (b, j, 0)),
            pl.BlockSpec((None, 3, N2), lambda b, j: (b, 0, 0)),
            pl.BlockSpec((None, RB, C1), lambda b, j: (b, j, 0)),
            pl.BlockSpec((None, N2, C2), lambda b, j: (b, 0, 0)),
            pl.BlockSpec((C1 + C2, H), lambda b, j: (0, 0)),
            pl.BlockSpec((1, H), lambda b, j: (0, 0)),
            pl.BlockSpec((H, O), lambda b, j: (0, 0)),
            pl.BlockSpec((1, O), lambda b, j: (0, 0)),
        ],
        out_specs=pl.BlockSpec((None, RB, O), lambda b, j: (b, j, 0)),
        out_shape=jax.ShapeDtypeStruct((B, N1, O), jnp.float32),
        scratch_shapes=[pltpu.VMEM((N2, H), jnp.float32)],
    )(xyz1, x2t, points1, points2, W1, b1r, W2, b2r)
    return out
